# baseline (speedup 1.0000x reference)
import functools

import numpy as np
import jax
import jax.numpy as jnp
from jax import lax
from jax.experimental import pallas as pl
from jax.experimental.pallas import tpu as pltpu
from jax.experimental.pallas import tpu_sc as plsc

V = 151936
B = 64
NIDS = 200
NW = 8
WSZ = V // NW
NVREG = WSZ // 16
NBIN = 2048
SBIN = 2048
SSHIFT = 21
CCAP = 2048
MCAP = 512
FCAP = 128
IDPAD = 208

TOPK = 50
KEEP_MIN = 250
PEN = np.float32(1.1)
TOPP_LIM = np.float32(1.0 - 0.8)
NEG_INF = np.float32(-np.inf)
TINY = np.float32(np.finfo(np.float32).tiny)
LN2 = np.float32(0.6931471805599453)
SQRT2 = np.float32(1.4142135)


def _i32(x):
    return jnp.int32(x)


def _mono(bits):
    asr = lax.shift_right_arithmetic(bits, _i32(31))
    return lax.bitwise_xor(bits, lax.shift_right_logical(asr, _i32(1)))


def _mono_f(v):
    return _mono(lax.bitcast_convert_type(v, jnp.int32))


def _log_f32(x):
    bits = lax.bitcast_convert_type(x, jnp.int32)
    e = lax.bitwise_and(lax.shift_right_logical(bits, _i32(23)), _i32(0xFF)) - _i32(127)
    mb = lax.bitwise_or(lax.bitwise_and(bits, _i32(0x007FFFFF)), _i32(0x3F800000))
    m = lax.bitcast_convert_type(mb, jnp.float32)
    big = m > SQRT2
    m = jnp.where(big, m * np.float32(0.5), m)
    e = e + big.astype(jnp.int32)
    z = (m - np.float32(1.0)) / (m + np.float32(1.0))
    z2 = z * z
    p = jnp.full_like(z, np.float32(1.0 / 9.0))
    for c in (1.0 / 7.0, 1.0 / 5.0, 1.0 / 3.0, 1.0):
        p = p * z2 + np.float32(c)
    return e.astype(jnp.float32) * LN2 + np.float32(2.0) * z * p


def _gumbel_vec(flat_idx):
    k0 = _i32(0)
    k1 = _i32(42)
    k2 = lax.bitwise_xor(lax.bitwise_xor(k0, k1), _i32(0x1BD11BDA))
    rot0 = (13, 15, 26, 6)
    rot1 = (17, 29, 16, 24)

    def rotl(v, d):
        return lax.bitwise_or(
            lax.shift_left(v, _i32(d)), lax.shift_right_logical(v, _i32(32 - d)))

    def four(x0, x1, rots):
        for d in rots:
            x0 = x0 + x1
            x1 = lax.bitwise_xor(x0, rotl(x1, d))
        return x0, x1

    x0 = jnp.zeros_like(flat_idx) + k0
    x1 = flat_idx + k1
    x0, x1 = four(x0, x1, rot0)
    x0, x1 = x0 + k1, x1 + k2 + _i32(1)
    x0, x1 = four(x0, x1, rot1)
    x0, x1 = x0 + k2, x1 + k0 + _i32(2)
    x0, x1 = four(x0, x1, rot0)
    x0, x1 = x0 + k0, x1 + k1 + _i32(3)
    x0, x1 = four(x0, x1, rot1)
    x0, x1 = x0 + k1, x1 + k2 + _i32(4)
    x0, x1 = four(x0, x1, rot0)
    x0, x1 = x0 + k2, x1 + k0 + _i32(5)
    bits = lax.bitwise_xor(x0, x1)
    fb = lax.bitwise_or(lax.shift_right_logical(bits, _i32(9)), _i32(0x3F800000))
    f = lax.bitcast_convert_type(fb, jnp.float32) - np.float32(1.0)
    u = jnp.maximum(TINY, f + TINY)
    w = -_log_f32(u)
    return -_log_f32(w)


class _RealOps:
    def fori(self, lo, hi, body, init):
        return lax.fori_loop(lo, hi, body, init)

    def vload(self, ref, start):
        return ref[pl.ds(start, 16)]

    def vstore(self, ref, start, v):
        ref[pl.ds(start, 16)] = v

    def sload(self, ref, i):
        idx = jnp.zeros((16,), jnp.int32) + i
        msk = lax.iota(jnp.int32, 16) == _i32(0)
        return plsc.load_gather(ref, [idx], mask=msk)[0]

    def popcnt(self, mask):
        return plsc.all_reduce_population_count(mask)[0]

    def pfori(self, lo, hi, body, init, unroll):
        if init is None:
            plsc.parallel_loop(lo, hi, unroll=unroll)(lambda i: body(i, None))
            return None
        return plsc.parallel_loop(lo, hi, unroll=unroll, carry=init)(body)

    def when(self, cond, fn):
        pl.when(cond)(fn)

    def sstore(self, ref, i, x):
        idx = jnp.zeros((16,), jnp.int32) + i
        xv = jnp.zeros((16,), jnp.asarray(x).dtype) + x
        msk = lax.iota(jnp.int32, 16) == _i32(0)
        plsc.store_scatter(ref, [idx], xv, mask=msk)

    def vgather(self, ref, idx):
        return plsc.load_gather(ref, [idx])

    def vscatter(self, ref, idx, x, mask=None):
        plsc.store_scatter(ref, [idx], x, mask=mask)

    def vscatter_add(self, ref, idx, x):
        plsc.addupdate_scatter(ref, [idx], x)

    def vcompress(self, ref, start, x, mask):
        plsc.store_compressed(ref.at[pl.ds(start, 16)], x, mask=mask)

    def copy_in(self, hbm, off, size, vmem):
        pltpu.sync_copy(hbm.at[pl.ds(off, size)], vmem.at[pl.ds(0, size)])

    def copy_start(self, hbm, off, size, vmem, sem):
        pltpu.make_async_copy(hbm.at[pl.ds(off, size)], vmem, sem).start()

    def copy_wait(self, hbm, off, size, vmem, sem):
        pltpu.make_async_copy(hbm.at[pl.ds(off, size)], vmem, sem).wait()

    def copy_out(self, vmem, hbm_row):
        pltpu.sync_copy(vmem, hbm_row)

    def gather_hbm(self, hbm, idx_ref, dst_ref, sem):
        pltpu.async_copy(hbm.at[idx_ref], dst_ref, sem).wait()


def _ufori(ops, n, body, init, unroll):
    main = n // unroll
    if main > 1:
        def mb(i, c):
            for u in range(unroll):
                c = body(i * unroll + u, c)
            return c

        c = ops.fori(0, main, mb, init)
    else:
        main = 0
        c = init
    for k in range(main * unroll, n):
        c = body(_i32(k), c)
    return c


def _stream_windows(ops, scores, rowbase, win0, win1, sem0, sem1,
                    make_body, init, unroll):
    bufs = (win0, win1)
    sems = (sem0, sem1)
    ops.copy_start(scores, rowbase, WSZ, bufs[0], sems[0])
    c = init
    for w in range(NW):
        b, sm = bufs[w % 2], sems[w % 2]
        ops.copy_wait(scores, rowbase + w * WSZ, WSZ, b, sm)
        if w + 1 < NW:
            ops.copy_start(scores, rowbase + (w + 1) * WSZ, WSZ,
                           bufs[(w + 1) % 2], sems[(w + 1) % 2])
        c = ops.pfori(0, NVREG, make_body(b, w), c, unroll)
    return c


def _row_program(ops, r, slot, scores, ids, refs):
    (win0, win1, lanehist, hist, bitmap, cval, cidx, cbin,
     idbuf, idflat, idval, idpen, idisc,
     member, fval, fidx, fp, fg, res, sem0, sem1, semg) = refs

    iota = lax.iota(jnp.int32, 16)
    zeros16 = jnp.zeros((16,), jnp.int32)
    ones16 = jnp.ones((16,), jnp.int32)
    ninf16 = jnp.full((16,), NEG_INF, jnp.float32)
    rowbase = r * _i32(V)

    def clr(i, c):
        ops.vstore(lanehist, i * 16, zeros16)
        return c

    ops.pfori(0, (16 * SBIN) // 16, clr, _i32(0), 8)

    laneoff = iota * _i32(SBIN)

    def hist_body(buf, w):
        def hist_vreg(k, c2):
            v = ops.vload(buf, k * 16)
            s = _mono_f(v)
            bn = lax.shift_right_arithmetic(s, _i32(SSHIFT)) + _i32(SBIN // 2)
            ops.vscatter_add(lanehist, bn + laneoff, ones16)
            return c2

        return hist_vreg

    _stream_windows(ops, scores, rowbase, win0, win1, sem0, sem1,
                    hist_body, _i32(0), 8)

    limit = _i32(V - KEEP_MIN)

    def merge(j, carry):
        run, cnt = carry
        acc = zeros16
        for l in range(16):
            acc = acc + ops.vload(lanehist, l * SBIN + j * 16)
        inc = jnp.cumsum(acc)
        pexc = run + (inc - acc)
        cnt = cnt + ops.popcnt(pexc <= limit)
        run = run + inc[15]
        return run, cnt

    _, cnt = ops.pfori(0, SBIN // 16, merge, (_i32(0), _i32(0)), 2)
    tau_s = lax.shift_left((cnt - _i32(1)) - _i32(SBIN // 2), _i32(SSHIFT))
    tauv = jnp.zeros((16,), jnp.int32) + tau_s
    taub = jnp.where(tauv >= _i32(0), tauv,
                     lax.bitwise_xor(tauv, _i32(0x7FFFFFFF)))
    tau_f = lax.bitcast_convert_type(taub, jnp.float32)[0]

    def ext_body(buf, w):
        def ext_vreg(k, ptr2):
            v = ops.vload(buf, k * 16)
            msk = v >= tau_f
            cnt2 = ops.popcnt(msk)
            idxv = _i32(w * WSZ) + k * 16 + iota
            ops.vcompress(cval, ptr2, v, msk)
            ops.vcompress(cidx, ptr2, idxv, msk)
            return jnp.minimum(ptr2 + cnt2, _i32(CCAP - 17))

        return ext_vreg

    nc = _stream_windows(ops, scores, rowbase, win0, win1, sem0, sem1,
                         ext_body, _i32(0), 16)
    ops.vscatter(cval, nc + iota, ninf16)
    ops.vscatter(cidx, nc + iota, zeros16)

    ops.copy_in(ids, r * _i32(NIDS), NIDS, idbuf)

    def mkflat(j, c):
        vv = ops.vload(idbuf, j * 16)
        valid = (j * 16 + iota) < _i32(NIDS)
        vv = jnp.where(valid, vv, _i32(0))
        ops.vstore(idbuf, j * 16, vv)
        ops.vstore(idflat, j * 16, rowbase + vv)
        return c

    ops.pfori(0, IDPAD // 16, mkflat, _i32(0), 4)
    ops.gather_hbm(scores, idflat, idval, semg)

    def idprep(j, c):
        v = ops.vload(idval, j * 16)
        pv = jnp.where(v < np.float32(0.0), v * PEN, v / PEN)
        ops.vstore(idpen, j * 16, pv)
        ops.vstore(idisc, j * 16, (v >= tau_f).astype(jnp.int32))
        return c

    ops.pfori(0, IDPAD // 16, idprep, _i32(0), 4)

    def bclr(i, c):
        ops.vstore(bitmap, i * 16, zeros16)
        return c

    ops.pfori(0, 4752 // 16, bclr, _i32(0), 8)

    def mark(j, c):
        idx = ops.sload(idbuf, j)
        w_ = lax.shift_right_logical(idx, _i32(5))
        bit = lax.shift_left(_i32(1), lax.bitwise_and(idx, _i32(31)))
        word = ops.sload(bitmap, w_)
        ops.sstore(bitmap, w_, lax.bitwise_or(word, bit))
        return c

    ops.fori(0, NIDS, mark, _i32(0))

    def cpen(j, c):
        idxv = ops.vload(cidx, j * 16)
        w_ = lax.shift_right_logical(idxv, _i32(5))
        words = ops.vgather(bitmap, w_)
        hit = lax.bitwise_and(
            lax.shift_right_logical(words, lax.bitwise_and(idxv, _i32(31))),
            _i32(1)) == _i32(1)
        v = ops.vload(cval, j * 16)
        pv = jnp.where(v < np.float32(0.0), v * PEN, v / PEN)
        ops.vstore(cval, j * 16, jnp.where(hit, pv, v))
        return c

    ops.pfori(0, (nc + _i32(15)) // 16, cpen, _i32(0), 4)

    def app(j, p):
        idx = ops.sload(idbuf, j)
        w_ = lax.shift_right_logical(idx, _i32(5))
        bit = lax.shift_left(_i32(1), lax.bitwise_and(idx, _i32(31)))
        word = ops.sload(bitmap, w_)
        first = lax.bitwise_and(word, bit) != _i32(0)
        ops.sstore(bitmap, w_, lax.bitwise_and(word, lax.bitwise_not(bit)))
        keep = jnp.logical_and(first, ops.sload(idisc, j) == _i32(0))
        ops.sstore(cval, p, ops.sload(idpen, j))
        ops.sstore(cidx, p, idx)
        return jnp.minimum(p + keep.astype(jnp.int32), _i32(CCAP - 17))

    m = ops.fori(0, NIDS, app, nc)
    ops.vscatter(cval, m + iota, ninf16)
    ops.vscatter(cidx, m + iota, zeros16)
    mchunks = (m + _i32(15)) // 16

    def hclr(i, c):
        ops.vstore(hist, i * 16, zeros16)
        return c

    ops.pfori(0, NBIN // 16, hclr, _i32(0), 8)

    def cbins(j, c):
        v = ops.vload(cval, j * 16)
        s = _mono_f(v)
        bn = lax.shift_right_arithmetic(s, _i32(21)) + _i32(1024)
        ops.vstore(cbin, j * 16, bn)
        return c

    ops.pfori(0, mchunks, cbins, _i32(0), 4)

    def chist(i, c):
        bn = ops.sload(cbin, i)
        ops.sstore(hist, bn, ops.sload(hist, bn) + _i32(1))
        return c

    ops.fori(0, m, chist, _i32(0))

    limit2 = m - _i32(TOPK)

    def scan2(j, carry):
        run, cnt2 = carry
        acc = ops.vload(hist, j * 16)
        inc = jnp.cumsum(acc)
        pexc = run + (inc - acc)
        cnt2 = cnt2 + ops.popcnt(pexc <= limit2)
        run = run + inc[15]
        return run, cnt2

    _, cnt2 = ops.pfori(0, NBIN // 16, scan2, (_i32(0), _i32(0)), 2)
    b50 = cnt2 - _i32(1)

    def mext(j, carry):
        nhi, q = carry
        bn = ops.vload(cbin, j * 16)
        lanemask = (j * 16 + iota) < m
        nhi = nhi + ops.popcnt(jnp.logical_and(bn > b50, lanemask))
        mm = jnp.logical_and(bn == b50, lanemask)
        cnt = ops.popcnt(mm)
        ops.vcompress(member, q, ops.vload(cval, j * 16), mm)
        q = jnp.minimum(q + cnt, _i32(MCAP - 17))
        return nhi, q

    c_hi, mb = ops.pfori(0, mchunks, mext, (_i32(0), _i32(0)), 2)
    ops.vscatter(member, mb + iota, ninf16)
    mbchunks = (mb + _i32(15)) // 16
    rank_lim = _i32(TOPK - 1) - c_hi

    def tsel(i, t):
        vi = ops.sload(member, i)

        def gcnt(j, g):
            mv = ops.vload(member, j * 16)
            return g + ops.popcnt(mv > vi)

        gt = ops.fori(0, mbchunks, gcnt, _i32(0))
        elig = gt <= rank_lim
        return jnp.where(elig, jnp.minimum(t, vi), t)

    t = ops.fori(0, mb, tsel, jnp.float32(np.inf))

    def fext(j, q):
        v = ops.vload(cval, j * 16)
        lanemask = (j * 16 + iota) < m
        mm = jnp.logical_and(v >= t, lanemask)
        cnt = ops.popcnt(mm)
        ops.vcompress(fval, q, v, mm)
        ops.vcompress(fidx, q, ops.vload(cidx, j * 16), mm)
        return jnp.minimum(q + cnt, _i32(FCAP - 17))

    nf = ops.pfori(0, mchunks, fext, _i32(0), 2)
    ops.vscatter(fval, nf + iota, ninf16)
    ops.vscatter(fidx, nf + iota, jnp.full((16,), _i32(1 << 29), jnp.int32))
    nfchunks = (nf + _i32(15)) // 16

    def vmax(j, mx):
        return jnp.maximum(mx, jnp.max(ops.vload(fval, j * 16)))

    mx = ops.pfori(0, nfchunks, vmax, jnp.float32(NEG_INF), 2)

    def lastidx(j, li):
        v = ops.vload(fval, j * 16)
        ii = ops.vload(fidx, j * 16)
        cand = jnp.where(v == mx, ii, _i32(-1))
        return jnp.maximum(li, jnp.max(cand))

    li = ops.pfori(0, nfchunks, lastidx, _i32(-1), 2)

    def esum(j, z):
        v = ops.vload(fval, j * 16)
        e = jnp.where(v == NEG_INF, np.float32(0.0), jnp.exp(v - mx))
        ops.vstore(fp, j * 16, e)
        return z + jnp.sum(e)

    z = ops.pfori(0, nfchunks, esum, jnp.float32(0.0), 2)

    def pnorm(j, c):
        ops.vstore(fp, j * 16, ops.vload(fp, j * 16) / z)
        return c

    ops.pfori(0, nfchunks, pnorm, _i32(0), 2)

    def gumb(j, c):
        ii = ops.vload(fidx, j * 16)
        safe = jnp.where(ii == _i32(1 << 29), _i32(0), ii)
        ops.vstore(fg, j * 16, _gumbel_vec(rowbase + safe))
        return c

    ops.pfori(0, nfchunks, gumb, _i32(0), 2)

    def pick(i, carry):
        bv, bi = carry
        vi = ops.sload(fval, i)
        ii = ops.sload(fidx, i)

        def csum(j, cacc):
            fv_j = ops.vload(fval, j * 16)
            fi_j = ops.vload(fidx, j * 16)
            p_j = ops.vload(fp, j * 16)
            earlier = jnp.logical_or(
                fv_j < vi, jnp.logical_and(fv_j == vi, fi_j <= ii))
            return cacc + jnp.sum(jnp.where(earlier, p_j, np.float32(0.0)))

        cum = ops.fori(0, nfchunks, csum, np.float32(0.0))
        is_last = jnp.logical_and(vi == mx, ii == li)
        removed = jnp.logical_and(cum <= TOPP_LIM, jnp.logical_not(is_last))
        gi = ops.sload(fg, i)
        tot = jnp.where(removed, NEG_INF, vi + gi)
        better = jnp.logical_or(
            tot > bv, jnp.logical_and(tot == bv, ii < bi))
        bv = jnp.where(better, tot, bv)
        bi = jnp.where(better, ii, bi)
        return bv, bi

    _, winner = ops.fori(0, nf, pick, (NEG_INF, _i32(1 << 29)))
    ops.sstore(res, slot, winner)


def _tec_body(ids_hbm, scores_hbm, out_hbm, *refs):
    wid = lax.axis_index("c") * _i32(16) + lax.axis_index("s")
    ops = _RealOps()
    res = refs[18]
    res[...] = jnp.zeros((16,), jnp.int32)
    for slot in range(2):
        r = wid * _i32(2) + _i32(slot)
        _row_program(ops, r, slot, scores_hbm, ids_hbm, refs)
    ops.copy_out(res, out_hbm.at[wid])


@jax.jit
def _run(ids_flat, scores_flat):
    mesh = plsc.VectorSubcoreMesh(core_axis_name="c", subcore_axis_name="s")
    scratch = [
        pltpu.VMEM((WSZ,), jnp.float32),
        pltpu.VMEM((WSZ,), jnp.float32),
        pltpu.VMEM((16 * SBIN,), jnp.int32),
        pltpu.VMEM((NBIN,), jnp.int32),
        pltpu.VMEM((4752,), jnp.int32),
        pltpu.VMEM((CCAP,), jnp.float32),
        pltpu.VMEM((CCAP,), jnp.int32),
        pltpu.VMEM((CCAP,), jnp.int32),
        pltpu.VMEM((IDPAD,), jnp.int32),
        pltpu.VMEM((IDPAD,), jnp.int32),
        pltpu.VMEM((IDPAD,), jnp.float32),
        pltpu.VMEM((IDPAD,), jnp.float32),
        pltpu.VMEM((IDPAD,), jnp.int32),
        pltpu.VMEM((MCAP,), jnp.float32),
        pltpu.VMEM((FCAP,), jnp.float32),
        pltpu.VMEM((FCAP,), jnp.int32),
        pltpu.VMEM((FCAP,), jnp.float32),
        pltpu.VMEM((FCAP,), jnp.float32),
        pltpu.VMEM((16,), jnp.int32),
        pltpu.SemaphoreType.DMA,
        pltpu.SemaphoreType.DMA,
        pltpu.SemaphoreType.DMA,
    ]
    f = pl.kernel(
        _tec_body,
        out_type=jax.ShapeDtypeStruct((32, 16), jnp.int32),
        mesh=mesh,
        scratch_types=scratch,
        compiler_params=pltpu.CompilerParams(needs_layout_passes=False),
    )
    return f(ids_flat, scores_flat)


def kernel(all_input_ids, logits):
    scores = logits[:, -1, :].reshape(-1)
    ids = all_input_ids.astype(jnp.int32).reshape(-1)
    out = _run(ids, scores)
    return out[:, :2].reshape(B)


# speedup vs baseline: 1.0049x; 1.0049x over previous
import functools

import numpy as np
import jax
import jax.numpy as jnp
from jax import lax
from jax.experimental import pallas as pl
from jax.experimental.pallas import tpu as pltpu
from jax.experimental.pallas import tpu_sc as plsc

V = 151936
B = 64
NIDS = 200
NW = 8
WSZ = V // NW
NVREG = WSZ // 16
NBIN = 2048
SBIN = 2048
SSHIFT = 21
CCAP = 2048
MCAP = 512
FCAP = 128
IDPAD = 208

TOPK = 50
KEEP_MIN = 250
PEN = np.float32(1.1)
TOPP_LIM = np.float32(1.0 - 0.8)
NEG_INF = np.float32(-np.inf)
TINY = np.float32(np.finfo(np.float32).tiny)
LN2 = np.float32(0.6931471805599453)
SQRT2 = np.float32(1.4142135)


def _i32(x):
    return jnp.int32(x)


def _mono(bits):
    asr = lax.shift_right_arithmetic(bits, _i32(31))
    return lax.bitwise_xor(bits, lax.shift_right_logical(asr, _i32(1)))


def _mono_f(v):
    return _mono(lax.bitcast_convert_type(v, jnp.int32))


def _log_f32(x):
    bits = lax.bitcast_convert_type(x, jnp.int32)
    e = lax.bitwise_and(lax.shift_right_logical(bits, _i32(23)), _i32(0xFF)) - _i32(127)
    mb = lax.bitwise_or(lax.bitwise_and(bits, _i32(0x007FFFFF)), _i32(0x3F800000))
    m = lax.bitcast_convert_type(mb, jnp.float32)
    big = m > SQRT2
    m = jnp.where(big, m * np.float32(0.5), m)
    e = e + big.astype(jnp.int32)
    z = (m - np.float32(1.0)) / (m + np.float32(1.0))
    z2 = z * z
    p = jnp.full_like(z, np.float32(1.0 / 9.0))
    for c in (1.0 / 7.0, 1.0 / 5.0, 1.0 / 3.0, 1.0):
        p = p * z2 + np.float32(c)
    return e.astype(jnp.float32) * LN2 + np.float32(2.0) * z * p


def _gumbel_vec(flat_idx):
    k0 = _i32(0)
    k1 = _i32(42)
    k2 = lax.bitwise_xor(lax.bitwise_xor(k0, k1), _i32(0x1BD11BDA))
    rot0 = (13, 15, 26, 6)
    rot1 = (17, 29, 16, 24)

    def rotl(v, d):
        return lax.bitwise_or(
            lax.shift_left(v, _i32(d)), lax.shift_right_logical(v, _i32(32 - d)))

    def four(x0, x1, rots):
        for d in rots:
            x0 = x0 + x1
            x1 = lax.bitwise_xor(x0, rotl(x1, d))
        return x0, x1

    x0 = jnp.zeros_like(flat_idx) + k0
    x1 = flat_idx + k1
    x0, x1 = four(x0, x1, rot0)
    x0, x1 = x0 + k1, x1 + k2 + _i32(1)
    x0, x1 = four(x0, x1, rot1)
    x0, x1 = x0 + k2, x1 + k0 + _i32(2)
    x0, x1 = four(x0, x1, rot0)
    x0, x1 = x0 + k0, x1 + k1 + _i32(3)
    x0, x1 = four(x0, x1, rot1)
    x0, x1 = x0 + k1, x1 + k2 + _i32(4)
    x0, x1 = four(x0, x1, rot0)
    x0, x1 = x0 + k2, x1 + k0 + _i32(5)
    bits = lax.bitwise_xor(x0, x1)
    fb = lax.bitwise_or(lax.shift_right_logical(bits, _i32(9)), _i32(0x3F800000))
    f = lax.bitcast_convert_type(fb, jnp.float32) - np.float32(1.0)
    u = jnp.maximum(TINY, f + TINY)
    w = -_log_f32(u)
    return -_log_f32(w)


class _RealOps:
    def fori(self, lo, hi, body, init):
        return lax.fori_loop(lo, hi, body, init)

    def vload(self, ref, start):
        return ref[pl.ds(start, 16)]

    def vstore(self, ref, start, v):
        ref[pl.ds(start, 16)] = v

    def sload(self, ref, i):
        idx = jnp.zeros((16,), jnp.int32) + i
        msk = lax.iota(jnp.int32, 16) == _i32(0)
        return plsc.load_gather(ref, [idx], mask=msk)[0]

    def popcnt(self, mask):
        return plsc.all_reduce_population_count(mask)[0]

    def pfori(self, lo, hi, body, init, unroll):
        if init is None:
            plsc.parallel_loop(lo, hi, unroll=unroll)(lambda i: body(i, None))
            return None
        return plsc.parallel_loop(lo, hi, unroll=unroll, carry=init)(body)

    def when(self, cond, fn):
        pl.when(cond)(fn)

    def sstore(self, ref, i, x):
        idx = jnp.zeros((16,), jnp.int32) + i
        xv = jnp.zeros((16,), jnp.asarray(x).dtype) + x
        msk = lax.iota(jnp.int32, 16) == _i32(0)
        plsc.store_scatter(ref, [idx], xv, mask=msk)

    def vgather(self, ref, idx):
        return plsc.load_gather(ref, [idx])

    def vscatter(self, ref, idx, x, mask=None):
        plsc.store_scatter(ref, [idx], x, mask=mask)

    def vscatter_add(self, ref, idx, x):
        plsc.addupdate_scatter(ref, [idx], x)

    def vcompress(self, ref, start, x, mask):
        plsc.store_compressed(ref.at[pl.ds(start, 16)], x, mask=mask)

    def copy_in(self, hbm, off, size, vmem):
        pltpu.sync_copy(hbm.at[pl.ds(off, size)], vmem.at[pl.ds(0, size)])

    def copy_start(self, hbm, off, size, vmem, sem):
        pltpu.make_async_copy(hbm.at[pl.ds(off, size)], vmem, sem).start()

    def copy_wait(self, hbm, off, size, vmem, sem):
        pltpu.make_async_copy(hbm.at[pl.ds(off, size)], vmem, sem).wait()

    def copy_out(self, vmem, hbm_row):
        pltpu.sync_copy(vmem, hbm_row)

    def gather_hbm(self, hbm, idx_ref, dst_ref, sem):
        pltpu.async_copy(hbm.at[idx_ref], dst_ref, sem).wait()


def _ufori(ops, n, body, init, unroll):
    main = n // unroll
    if main > 1:
        def mb(i, c):
            for u in range(unroll):
                c = body(i * unroll + u, c)
            return c

        c = ops.fori(0, main, mb, init)
    else:
        main = 0
        c = init
    for k in range(main * unroll, n):
        c = body(_i32(k), c)
    return c


def _stream_windows(ops, scores, rowbase, win0, win1, sem0, sem1,
                    make_body, init, unroll):
    bufs = (win0, win1)
    sems = (sem0, sem1)
    ops.copy_start(scores, rowbase, WSZ, bufs[0], sems[0])
    c = init
    for w in range(NW):
        b, sm = bufs[w % 2], sems[w % 2]
        ops.copy_wait(scores, rowbase + w * WSZ, WSZ, b, sm)
        if w + 1 < NW:
            ops.copy_start(scores, rowbase + (w + 1) * WSZ, WSZ,
                           bufs[(w + 1) % 2], sems[(w + 1) % 2])
        c = ops.pfori(0, NVREG, make_body(b, w), c, unroll)
    return c


def _row_program(ops, r, slot, scores, ids, refs):
    (win0, win1, lanehist, hist, bitmap, cval, cidx, cbin,
     idbuf, idflat, idval, idpen, idisc,
     member, fval, fidx, fp, fg, res, sem0, sem1, semg) = refs

    iota = lax.iota(jnp.int32, 16)
    zeros16 = jnp.zeros((16,), jnp.int32)
    ones16 = jnp.ones((16,), jnp.int32)
    ninf16 = jnp.full((16,), NEG_INF, jnp.float32)
    rowbase = r * _i32(V)

    def clr(i, c):
        ops.vstore(lanehist, i * 16, zeros16)
        return c

    ops.pfori(0, (16 * SBIN) // 16, clr, _i32(0), 8)

    laneoff = iota * _i32(SBIN)

    def hist_body(buf, w):
        def hist_vreg(k, c2):
            v = ops.vload(buf, k * 16)
            s = _mono_f(v)
            bn = lax.shift_right_arithmetic(s, _i32(SSHIFT)) + _i32(SBIN // 2)
            ops.vscatter_add(lanehist, bn + laneoff, ones16)
            return c2

        return hist_vreg

    _stream_windows(ops, scores, rowbase, win0, win1, sem0, sem1,
                    hist_body, _i32(0), 8)

    limit = _i32(V - KEEP_MIN)

    def merge(j, carry):
        run, cnt = carry
        acc = zeros16
        for l in range(16):
            acc = acc + ops.vload(lanehist, l * SBIN + j * 16)
        inc = jnp.cumsum(acc)
        pexc = run + (inc - acc)
        cnt = cnt + ops.popcnt(pexc <= limit)
        run = run + inc[15]
        return run, cnt

    _, cnt = ops.pfori(0, SBIN // 16, merge, (_i32(0), _i32(0)), 2)
    tau_s = lax.shift_left((cnt - _i32(1)) - _i32(SBIN // 2), _i32(SSHIFT))
    tauv = jnp.zeros((16,), jnp.int32) + tau_s
    taub = jnp.where(tauv >= _i32(0), tauv,
                     lax.bitwise_xor(tauv, _i32(0x7FFFFFFF)))
    tau_f = lax.bitcast_convert_type(taub, jnp.float32)[0]

    def ext_body(buf, w):
        def ext_vreg(k, ptr2):
            v = ops.vload(buf, k * 16)
            msk = v >= tau_f
            cnt2 = ops.popcnt(msk)
            idxv = _i32(w * WSZ) + k * 16 + iota
            ops.vcompress(cval, ptr2, v, msk)
            ops.vcompress(cidx, ptr2, idxv, msk)
            return jnp.minimum(ptr2 + cnt2, _i32(CCAP - 17))

        return ext_vreg

    nc = _stream_windows(ops, scores, rowbase, win0, win1, sem0, sem1,
                         ext_body, _i32(0), 8)
    ops.vscatter(cval, nc + iota, ninf16)
    ops.vscatter(cidx, nc + iota, zeros16)

    ops.copy_in(ids, r * _i32(NIDS), NIDS, idbuf)

    def mkflat(j, c):
        vv = ops.vload(idbuf, j * 16)
        valid = (j * 16 + iota) < _i32(NIDS)
        vv = jnp.where(valid, vv, _i32(0))
        ops.vstore(idbuf, j * 16, vv)
        ops.vstore(idflat, j * 16, rowbase + vv)
        return c

    ops.pfori(0, IDPAD // 16, mkflat, _i32(0), 4)
    ops.gather_hbm(scores, idflat, idval, semg)

    def idprep(j, c):
        v = ops.vload(idval, j * 16)
        pv = jnp.where(v < np.float32(0.0), v * PEN, v / PEN)
        ops.vstore(idpen, j * 16, pv)
        ops.vstore(idisc, j * 16, (v >= tau_f).astype(jnp.int32))
        return c

    ops.pfori(0, IDPAD // 16, idprep, _i32(0), 4)

    def bclr(i, c):
        ops.vstore(bitmap, i * 16, zeros16)
        return c

    ops.pfori(0, 4752 // 16, bclr, _i32(0), 8)

    def mark(j, c):
        idx = ops.sload(idbuf, j)
        w_ = lax.shift_right_logical(idx, _i32(5))
        bit = lax.shift_left(_i32(1), lax.bitwise_and(idx, _i32(31)))
        word = ops.sload(bitmap, w_)
        ops.sstore(bitmap, w_, lax.bitwise_or(word, bit))
        return c

    ops.fori(0, NIDS, mark, _i32(0))

    def cpen(j, c):
        idxv = ops.vload(cidx, j * 16)
        w_ = lax.shift_right_logical(idxv, _i32(5))
        words = ops.vgather(bitmap, w_)
        hit = lax.bitwise_and(
            lax.shift_right_logical(words, lax.bitwise_and(idxv, _i32(31))),
            _i32(1)) == _i32(1)
        v = ops.vload(cval, j * 16)
        pv = jnp.where(v < np.float32(0.0), v * PEN, v / PEN)
        ops.vstore(cval, j * 16, jnp.where(hit, pv, v))
        return c

    ops.pfori(0, (nc + _i32(15)) // 16, cpen, _i32(0), 4)

    def app(j, p):
        idx = ops.sload(idbuf, j)
        w_ = lax.shift_right_logical(idx, _i32(5))
        bit = lax.shift_left(_i32(1), lax.bitwise_and(idx, _i32(31)))
        word = ops.sload(bitmap, w_)
        first = lax.bitwise_and(word, bit) != _i32(0)
        ops.sstore(bitmap, w_, lax.bitwise_and(word, lax.bitwise_not(bit)))
        keep = jnp.logical_and(first, ops.sload(idisc, j) == _i32(0))
        ops.sstore(cval, p, ops.sload(idpen, j))
        ops.sstore(cidx, p, idx)
        return jnp.minimum(p + keep.astype(jnp.int32), _i32(CCAP - 17))

    m = ops.fori(0, NIDS, app, nc)
    ops.vscatter(cval, m + iota, ninf16)
    ops.vscatter(cidx, m + iota, zeros16)
    mchunks = (m + _i32(15)) // 16

    def hclr(i, c):
        ops.vstore(hist, i * 16, zeros16)
        return c

    ops.pfori(0, NBIN // 16, hclr, _i32(0), 8)

    def cbins(j, c):
        v = ops.vload(cval, j * 16)
        s = _mono_f(v)
        bn = lax.shift_right_arithmetic(s, _i32(21)) + _i32(1024)
        ops.vstore(cbin, j * 16, bn)
        return c

    ops.pfori(0, mchunks, cbins, _i32(0), 4)

    def chist(i, c):
        bn = ops.sload(cbin, i)
        ops.sstore(hist, bn, ops.sload(hist, bn) + _i32(1))
        return c

    ops.fori(0, m, chist, _i32(0))

    limit2 = m - _i32(TOPK)

    def scan2(j, carry):
        run, cnt2 = carry
        acc = ops.vload(hist, j * 16)
        inc = jnp.cumsum(acc)
        pexc = run + (inc - acc)
        cnt2 = cnt2 + ops.popcnt(pexc <= limit2)
        run = run + inc[15]
        return run, cnt2

    _, cnt2 = ops.pfori(0, NBIN // 16, scan2, (_i32(0), _i32(0)), 2)
    b50 = cnt2 - _i32(1)

    def mext(j, carry):
        nhi, q = carry
        bn = ops.vload(cbin, j * 16)
        lanemask = (j * 16 + iota) < m
        nhi = nhi + ops.popcnt(jnp.logical_and(bn > b50, lanemask))
        mm = jnp.logical_and(bn == b50, lanemask)
        cnt = ops.popcnt(mm)
        ops.vcompress(member, q, ops.vload(cval, j * 16), mm)
        q = jnp.minimum(q + cnt, _i32(MCAP - 17))
        return nhi, q

    c_hi, mb = ops.pfori(0, mchunks, mext, (_i32(0), _i32(0)), 2)
    ops.vscatter(member, mb + iota, ninf16)
    mbchunks = (mb + _i32(15)) // 16
    rank_lim = _i32(TOPK - 1) - c_hi

    def tsel(i, t):
        vi = ops.sload(member, i)

        def gcnt(j, g):
            mv = ops.vload(member, j * 16)
            return g + ops.popcnt(mv > vi)

        gt = ops.fori(0, mbchunks, gcnt, _i32(0))
        elig = gt <= rank_lim
        return jnp.where(elig, jnp.minimum(t, vi), t)

    t = ops.fori(0, mb, tsel, jnp.float32(np.inf))

    def fext(j, q):
        v = ops.vload(cval, j * 16)
        lanemask = (j * 16 + iota) < m
        mm = jnp.logical_and(v >= t, lanemask)
        cnt = ops.popcnt(mm)
        ops.vcompress(fval, q, v, mm)
        ops.vcompress(fidx, q, ops.vload(cidx, j * 16), mm)
        return jnp.minimum(q + cnt, _i32(FCAP - 17))

    nf = ops.pfori(0, mchunks, fext, _i32(0), 2)
    ops.vscatter(fval, nf + iota, ninf16)
    ops.vscatter(fidx, nf + iota, jnp.full((16,), _i32(1 << 29), jnp.int32))
    nfchunks = (nf + _i32(15)) // 16

    def vmax(j, mx):
        return jnp.maximum(mx, jnp.max(ops.vload(fval, j * 16)))

    mx = ops.pfori(0, nfchunks, vmax, jnp.float32(NEG_INF), 2)

    def lastidx(j, li):
        v = ops.vload(fval, j * 16)
        ii = ops.vload(fidx, j * 16)
        cand = jnp.where(v == mx, ii, _i32(-1))
        return jnp.maximum(li, jnp.max(cand))

    li = ops.pfori(0, nfchunks, lastidx, _i32(-1), 2)

    def esum(j, z):
        v = ops.vload(fval, j * 16)
        e = jnp.where(v == NEG_INF, np.float32(0.0), jnp.exp(v - mx))
        ops.vstore(fp, j * 16, e)
        return z + jnp.sum(e)

    z = ops.pfori(0, nfchunks, esum, jnp.float32(0.0), 2)

    def pnorm(j, c):
        ops.vstore(fp, j * 16, ops.vload(fp, j * 16) / z)
        return c

    ops.pfori(0, nfchunks, pnorm, _i32(0), 2)

    def gumb(j, c):
        ii = ops.vload(fidx, j * 16)
        safe = jnp.where(ii == _i32(1 << 29), _i32(0), ii)
        ops.vstore(fg, j * 16, _gumbel_vec(rowbase + safe))
        return c

    ops.pfori(0, nfchunks, gumb, _i32(0), 2)

    def pick(i, carry):
        bv, bi = carry
        vi = ops.sload(fval, i)
        ii = ops.sload(fidx, i)

        def csum(j, cacc):
            fv_j = ops.vload(fval, j * 16)
            fi_j = ops.vload(fidx, j * 16)
            p_j = ops.vload(fp, j * 16)
            earlier = jnp.logical_or(
                fv_j < vi, jnp.logical_and(fv_j == vi, fi_j <= ii))
            return cacc + jnp.sum(jnp.where(earlier, p_j, np.float32(0.0)))

        cum = ops.fori(0, nfchunks, csum, np.float32(0.0))
        is_last = jnp.logical_and(vi == mx, ii == li)
        removed = jnp.logical_and(cum <= TOPP_LIM, jnp.logical_not(is_last))
        gi = ops.sload(fg, i)
        tot = jnp.where(removed, NEG_INF, vi + gi)
        better = jnp.logical_or(
            tot > bv, jnp.logical_and(tot == bv, ii < bi))
        bv = jnp.where(better, tot, bv)
        bi = jnp.where(better, ii, bi)
        return bv, bi

    _, winner = ops.fori(0, nf, pick, (NEG_INF, _i32(1 << 29)))
    ops.sstore(res, slot, winner)


def _tec_body(ids_hbm, scores_hbm, out_hbm, *refs):
    wid = lax.axis_index("c") * _i32(16) + lax.axis_index("s")
    ops = _RealOps()
    res = refs[18]
    res[...] = jnp.zeros((16,), jnp.int32)
    for slot in range(2):
        r = wid * _i32(2) + _i32(slot)
        _row_program(ops, r, slot, scores_hbm, ids_hbm, refs)
    ops.copy_out(res, out_hbm.at[wid])


@jax.jit
def _run(ids_flat, scores_flat):
    mesh = plsc.VectorSubcoreMesh(core_axis_name="c", subcore_axis_name="s")
    scratch = [
        pltpu.VMEM((WSZ,), jnp.float32),
        pltpu.VMEM((WSZ,), jnp.float32),
        pltpu.VMEM((16 * SBIN,), jnp.int32),
        pltpu.VMEM((NBIN,), jnp.int32),
        pltpu.VMEM((4752,), jnp.int32),
        pltpu.VMEM((CCAP,), jnp.float32),
        pltpu.VMEM((CCAP,), jnp.int32),
        pltpu.VMEM((CCAP,), jnp.int32),
        pltpu.VMEM((IDPAD,), jnp.int32),
        pltpu.VMEM((IDPAD,), jnp.int32),
        pltpu.VMEM((IDPAD,), jnp.float32),
        pltpu.VMEM((IDPAD,), jnp.float32),
        pltpu.VMEM((IDPAD,), jnp.int32),
        pltpu.VMEM((MCAP,), jnp.float32),
        pltpu.VMEM((FCAP,), jnp.float32),
        pltpu.VMEM((FCAP,), jnp.int32),
        pltpu.VMEM((FCAP,), jnp.float32),
        pltpu.VMEM((FCAP,), jnp.float32),
        pltpu.VMEM((16,), jnp.int32),
        pltpu.SemaphoreType.DMA,
        pltpu.SemaphoreType.DMA,
        pltpu.SemaphoreType.DMA,
    ]
    f = pl.kernel(
        _tec_body,
        out_type=jax.ShapeDtypeStruct((32, 16), jnp.int32),
        mesh=mesh,
        scratch_types=scratch,
        compiler_params=pltpu.CompilerParams(needs_layout_passes=False),
    )
    return f(ids_flat, scores_flat)


def kernel(all_input_ids, logits):
    scores = logits[:, -1, :].reshape(-1)
    ids = all_input_ids.astype(jnp.int32).reshape(-1)
    out = _run(ids, scores)
    return out[:, :2].reshape(B)
